# baseline (speedup 1.0000x reference)
import functools

import jax
import jax.numpy as jnp
from jax import lax
from jax.experimental import pallas as pl
from jax.experimental.pallas import tpu as pltpu
from jax.experimental.pallas import tpu_sc as plsc

VOCAB = 1000000
DIM = 128
BATCH = 1024
SEQ = 200

NC = 2
NS = 16
NW = NC * NS

N = BATCH * SEQ
ROWS_PER_W = N // NW
CHUNK = 128
NCHUNK = ROWS_PER_W // CHUNK


NBUF = 5
NITER = NCHUNK // NBUF


WGROUPS = ((0, 2), (2, 2), (4, 1))


def _gather_kernel(table_hbm, idx_hbm, out_hbm, idx_v, buf, *sems):
    gsems = sems[:NBUF]
    osems = sems[NBUF:NBUF + len(WGROUPS)]
    wid = lax.axis_index("s") * NC + lax.axis_index("c")
    base = wid * ROWS_PER_W
    pltpu.sync_copy(idx_hbm.at[pl.ds(base, ROWS_PER_W)], idx_v)

    def fire_gather(c, j):
        pltpu.async_copy(table_hbm.at[idx_v.at[pl.ds(c * CHUNK, CHUNK)]],
                         buf.at[pl.ds(j * CHUNK, CHUNK)], gsems[j])

    def drain_gather(j):
        pltpu.make_async_copy(table_hbm.at[idx_v.at[pl.ds(0, CHUNK)]],
                              buf.at[pl.ds(j * CHUNK, CHUNK)], gsems[j]).wait()

    def fire_write(c0, g):
        j0, w = WGROUPS[g]
        pltpu.async_copy(buf.at[pl.ds(j0 * CHUNK, w * CHUNK)],
                         out_hbm.at[pl.ds(base + (c0 + j0) * CHUNK, w * CHUNK)],
                         osems[g])

    def drain_write(g):
        j0, w = WGROUPS[g]
        pltpu.make_async_copy(buf.at[pl.ds(j0 * CHUNK, w * CHUNK)],
                              out_hbm.at[pl.ds(base, w * CHUNK)],
                              osems[g]).wait()

    for j in range(NBUF):
        fire_gather(j, j)

    def body(t, _):
        c0 = t * NBUF

        def refill(g):
            j0, w = WGROUPS[g]

            @pl.when(t < NITER - 1)
            def _():
                drain_write(g)
                for j in range(j0, j0 + w):
                    fire_gather(c0 + NBUF + j, j)

        drain_gather(0)
        drain_gather(1)
        fire_write(c0, 0)
        drain_gather(2)
        drain_gather(3)
        fire_write(c0, 1)
        refill(0)
        drain_gather(4)
        fire_write(c0, 2)
        refill(1)
        refill(2)
        return 0

    lax.fori_loop(0, NITER, body, 0)
    for g in range(len(WGROUPS)):
        drain_write(g)


@jax.jit
def _gather(table, idx_flat):
    mesh = plsc.VectorSubcoreMesh(core_axis_name="c", subcore_axis_name="s")
    return pl.kernel(
        _gather_kernel,
        out_type=jax.ShapeDtypeStruct((N, DIM), jnp.float32),
        mesh=mesh,
        scratch_types=(
            [pltpu.VMEM((ROWS_PER_W,), jnp.int32)]
            + [pltpu.VMEM((NBUF * CHUNK, DIM), jnp.float32)]
            + [pltpu.SemaphoreType.DMA] * (NBUF + len(WGROUPS))
        ),
    )(table, idx_flat)


def kernel(input_ids, W, P, gamma, beta):
    idx_flat = input_ids.reshape(-1).astype(jnp.int32)
    out = _gather(W, idx_flat)
    return out.reshape(BATCH, SEQ, DIM)


# speedup vs baseline: 1.0158x; 1.0158x over previous
import functools

import jax
import jax.numpy as jnp
from jax import lax
from jax.experimental import pallas as pl
from jax.experimental.pallas import tpu as pltpu
from jax.experimental.pallas import tpu_sc as plsc

VOCAB = 1000000
DIM = 128
BATCH = 1024
SEQ = 200

NC = 2
NS = 16
NW = NC * NS

N = BATCH * SEQ
ROWS_PER_W = N // NW
CHUNK = 128
NCHUNK = ROWS_PER_W // CHUNK


NBUF = 5
NITER = NCHUNK // NBUF


def _gather_kernel(table_hbm, idx_hbm, out_hbm, idx_v, *sc):
    bufs = sc[:NBUF]
    gsems = sc[NBUF:2 * NBUF]
    osems = sc[2 * NBUF:3 * NBUF]
    wid = lax.axis_index("s") * NC + lax.axis_index("c")
    base = wid * ROWS_PER_W
    pltpu.sync_copy(idx_hbm.at[pl.ds(base, ROWS_PER_W)], idx_v)

    def fire_gather(c, j):
        pltpu.async_copy(table_hbm.at[idx_v.at[pl.ds(c * CHUNK, CHUNK)]],
                         bufs[j], gsems[j])

    def drain_gather(j):
        pltpu.make_async_copy(table_hbm.at[idx_v.at[pl.ds(0, CHUNK)]],
                              bufs[j], gsems[j]).wait()

    def fire_write(c, j):
        pltpu.async_copy(bufs[j], out_hbm.at[pl.ds(base + c * CHUNK, CHUNK)],
                         osems[j])

    def drain_write(j):
        pltpu.make_async_copy(bufs[j], out_hbm.at[pl.ds(base, CHUNK)],
                              osems[j]).wait()

    for j in range(NBUF):
        fire_gather(j, j)

    def body(t, _):
        c0 = t * NBUF

        def refill(j):
            @pl.when(t < NITER - 1)
            def _():
                drain_write(j)
                fire_gather(c0 + NBUF + j, j)

        for j in range(NBUF):
            drain_gather(j)
            fire_write(c0 + j, j)
            if j >= 1:
                refill(j - 1)
        refill(NBUF - 1)
        return 0

    lax.fori_loop(0, NITER, body, 0)
    for j in range(NBUF):
        drain_write(j)


@jax.jit
def _gather(table, idx_flat):
    mesh = plsc.VectorSubcoreMesh(core_axis_name="c", subcore_axis_name="s")
    return pl.kernel(
        _gather_kernel,
        out_type=jax.ShapeDtypeStruct((N, DIM), jnp.float32),
        mesh=mesh,
        scratch_types=(
            [pltpu.VMEM((ROWS_PER_W,), jnp.int32)]
            + [pltpu.VMEM((CHUNK, DIM), jnp.float32)] * NBUF
            + [pltpu.SemaphoreType.DMA] * (2 * NBUF)
        ),
    )(table, idx_flat)


def kernel(input_ids, W, P, gamma, beta):
    idx_flat = input_ids.reshape(-1).astype(jnp.int32)
    out = _gather(W, idx_flat)
    return out.reshape(BATCH, SEQ, DIM)


# speedup vs baseline: 1.0293x; 1.0133x over previous
import functools

import jax
import jax.numpy as jnp
from jax import lax
from jax.experimental import pallas as pl
from jax.experimental.pallas import tpu as pltpu
from jax.experimental.pallas import tpu_sc as plsc

VOCAB = 1000000
DIM = 128
BATCH = 1024
SEQ = 200

NC = 2
NS = 16
NW = NC * NS

N = BATCH * SEQ
ROWS_PER_W = N // NW
CHUNK = 64
NCHUNK = ROWS_PER_W // CHUNK


NBUF = 10
NITER = NCHUNK // NBUF


def _gather_kernel(table_hbm, idx_hbm, out_hbm, idx_v, *sc):
    bufs = sc[:NBUF]
    gsems = sc[NBUF:2 * NBUF]
    osems = sc[2 * NBUF:3 * NBUF]
    wid = lax.axis_index("s") * NC + lax.axis_index("c")
    base = wid * ROWS_PER_W
    pltpu.sync_copy(idx_hbm.at[pl.ds(base, ROWS_PER_W)], idx_v)

    def fire_gather(c, j):
        pltpu.async_copy(table_hbm.at[idx_v.at[pl.ds(c * CHUNK, CHUNK)]],
                         bufs[j], gsems[j])

    def drain_gather(j):
        pltpu.make_async_copy(table_hbm.at[idx_v.at[pl.ds(0, CHUNK)]],
                              bufs[j], gsems[j]).wait()

    def fire_write(c, j):
        pltpu.async_copy(bufs[j], out_hbm.at[pl.ds(base + c * CHUNK, CHUNK)],
                         osems[j])

    def drain_write(j):
        pltpu.make_async_copy(bufs[j], out_hbm.at[pl.ds(base, CHUNK)],
                              osems[j]).wait()

    for j in range(NBUF):
        fire_gather(j, j)

    def body(t, _):
        c0 = t * NBUF

        def refill(j):
            @pl.when(t < NITER - 1)
            def _():
                drain_write(j)
                fire_gather(c0 + NBUF + j, j)

        for j in range(NBUF):
            drain_gather(j)
            fire_write(c0 + j, j)
            if j >= 1:
                refill(j - 1)
        refill(NBUF - 1)
        return 0

    lax.fori_loop(0, NITER, body, 0)
    for j in range(NBUF):
        drain_write(j)


@jax.jit
def _gather(table, idx_flat):
    mesh = plsc.VectorSubcoreMesh(core_axis_name="c", subcore_axis_name="s")
    return pl.kernel(
        _gather_kernel,
        out_type=jax.ShapeDtypeStruct((N, DIM), jnp.float32),
        mesh=mesh,
        scratch_types=(
            [pltpu.VMEM((ROWS_PER_W,), jnp.int32)]
            + [pltpu.VMEM((CHUNK, DIM), jnp.float32)] * NBUF
            + [pltpu.SemaphoreType.DMA] * (2 * NBUF)
        ),
    )(table, idx_flat)


def kernel(input_ids, W, P, gamma, beta):
    idx_flat = input_ids.reshape(-1).astype(jnp.int32)
    out = _gather(W, idx_flat)
    return out.reshape(BATCH, SEQ, DIM)
